# Initial kernel scaffold; baseline (speedup 1.0000x reference)
import jax
import jax.numpy as jnp
from jax.experimental import pallas as pl


def kernel(x, params, edge_index, batch):
    raise NotImplementedError("write your pallas kernel here")


# speedup vs baseline: 7.5959x; 7.5959x over previous
import functools

import jax
import jax.numpy as jnp
from jax.experimental import pallas as pl
from jax.experimental.pallas import tpu as pltpu

HIDDEN = 64
HEADS = 8
HEAD_DIM = 8
BATCH = 16
NODE_DIM = 7
NUM_PATTERNS = 5
HUB_BIAS = 3.0
EXPL = 0.15 * (1.0 - 0.0)

_INTERPRET = jax.default_backend() == "cpu"


_BLK = 4096


def _colmax_body(x_ref, out_ref):
    i = pl.program_id(0)
    m = jnp.max(x_ref[...][:, 4:7], axis=0, keepdims=True)

    @pl.when(i == 0)
    def _():
        out_ref[...] = m

    @pl.when(i > 0)
    def _():
        out_ref[...] = jnp.maximum(out_ref[...], m)


def _hub_body(x_ref, cm_ref, out_ref):
    x = x_ref[...]
    cm = cm_ref[...]
    fan_in = x[:, 0:1]
    fan_out = x[:, 1:2]
    degc = x[:, 2:3]
    ratio = x[:, 3:4]
    degree_score = jax.nn.sigmoid(fan_in + fan_out - 3.0)
    out_degree_score = jax.nn.sigmoid(fan_out - 2.0)
    balance = jnp.clip(1.0 - jnp.abs(ratio - 1.0), 0.0, 1.0)
    pr_s = x[:, 4:5] / (cm[0, 0] + 1e-08)
    bt_s = x[:, 5:6] / (cm[0, 1] + 1e-08)
    cl_s = x[:, 6:7] / (cm[0, 2] + 1e-08)
    hub = (0.25 * degree_score + 0.2 * out_degree_score + 0.15 * balance
           + 0.15 * pr_s + 0.1 * bt_s + 0.1 * cl_s + 0.05 * degc)
    out_ref[...] = jnp.clip(hub, 0.0, 1.0)


def _hub_scores(x_pad):
    np_ = x_pad.shape[0]
    grid = np_ // _BLK if np_ % _BLK == 0 else pl.cdiv(np_, _BLK)
    cm = pl.pallas_call(
        _colmax_body,
        grid=(grid,),
        in_specs=[pl.BlockSpec((_BLK, NODE_DIM), lambda i: (i, 0))],
        out_specs=pl.BlockSpec((1, 3), lambda i: (0, 0)),
        out_shape=jax.ShapeDtypeStruct((1, 3), jnp.float32),
        interpret=_INTERPRET,
    )(x_pad)
    out = pl.pallas_call(
        _hub_body,
        grid=(grid,),
        in_specs=[
            pl.BlockSpec((_BLK, NODE_DIM), lambda i: (i, 0)),
            pl.BlockSpec((1, 3), lambda i: (0, 0)),
        ],
        out_specs=pl.BlockSpec((_BLK, 1), lambda i: (i, 0)),
        out_shape=jax.ShapeDtypeStruct((np_, 1), jnp.float32),
        interpret=_INTERPRET,
    )(x_pad, cm)
    return out[:, 0]


def _embed_body(x_ref, we_ref, be_ref, wg_ref, xemb_ref, xw_ref):
    xe = jnp.dot(x_ref[...], we_ref[...],
                 preferred_element_type=jnp.float32) + be_ref[...]
    xemb_ref[...] = xe
    xw_ref[...] = jnp.dot(xe, wg_ref[...], preferred_element_type=jnp.float32)


def _embed(x, W_embed, b_embed, W_g0):
    n = x.shape[0]
    blk = _BLK
    grid = pl.cdiv(n, blk)
    return pl.pallas_call(
        _embed_body,
        grid=(grid,),
        in_specs=[
            pl.BlockSpec((blk, NODE_DIM), lambda i: (i, 0)),
            pl.BlockSpec((NODE_DIM, HIDDEN), lambda i: (0, 0)),
            pl.BlockSpec((1, HIDDEN), lambda i: (0, 0)),
            pl.BlockSpec((HIDDEN, HIDDEN), lambda i: (0, 0)),
        ],
        out_specs=[
            pl.BlockSpec((blk, HIDDEN), lambda i: (i, 0)),
            pl.BlockSpec((blk, HIDDEN), lambda i: (i, 0)),
        ],
        out_shape=[
            jax.ShapeDtypeStruct((n, HIDDEN), jnp.float32),
            jax.ShapeDtypeStruct((n, HIDDEN), jnp.float32),
        ],
        interpret=_INTERPRET,
    )(x, W_embed, b_embed.reshape(1, HIDDEN), W_g0)


def _seg_sum(d, i, n):
    return jax.ops.segment_sum(d, i, num_segments=n)


def _seg_max(d, i, n):
    return jax.ops.segment_max(d, i, num_segments=n)


def _graph_norm(x, gamma, beta, mean_scale):
    mean = x.mean(axis=0, keepdims=True)
    centered = x - mean_scale * mean
    var = (centered * centered).mean(axis=0, keepdims=True)
    return gamma * centered / jnp.sqrt(var + 1e-05) + beta


def _top_hubs(hub_scores, batch, top_k, batch_size):
    mask = jnp.zeros(hub_scores.shape, dtype=bool)
    for b in range(batch_size):
        masked = jnp.where(batch == b, hub_scores, -jnp.inf)
        _, idx = jax.lax.top_k(masked, top_k)
        mask = mask.at[idx].set(True)
    return mask


def kernel(x, params, edge_index, batch):
    n = x.shape[0]
    p = params
    src = edge_index[0]
    dst = edge_index[1]

    structural = x[:, :NODE_DIM]
    n_pad = ((n + _BLK - 1) // _BLK) * _BLK
    x_pad = jnp.pad(x, ((0, n_pad - n), (0, 0)))
    hub_scores = _hub_scores(x_pad)[:n]
    top_hub_mask = _top_hubs(hub_scores, batch, 3, BATCH)

    x_emb, xw0 = _embed(x_pad, p['W_embed'], p['b_embed'], p['W_g0'])
    x_emb, xw0 = x_emb[:n], xw0[:n]

    deg = _seg_sum(jnp.ones(src.shape, jnp.float32), dst, n) + 1.0
    dinv = jax.lax.rsqrt(deg)

    y = xw0 * dinv[:, None]
    acc = _seg_sum(y[src], dst, n) + y
    h = dinv[:, None] * acc + p['b_g0']
    h = _graph_norm(h, p['gn0_g'], p['gn0_b'], p['gn0_a'])
    x_emb = jax.nn.relu(h) + x_emb

    xl = (x_emb @ p['W_a1']).reshape(n, HEADS, HEAD_DIM)
    a_src = (xl * p['att_src']).sum(-1)
    a_dst = (xl * p['att_dst']).sum(-1)
    alpha_e = jax.nn.leaky_relu(a_src[src] + a_dst[dst], 0.2)
    alpha_self = jax.nn.leaky_relu(a_src + a_dst, 0.2)
    m = jnp.maximum(_seg_max(alpha_e, dst, n), alpha_self)
    e = jnp.exp(alpha_e - m[dst])
    e_self = jnp.exp(alpha_self - m)
    denom = _seg_sum(e, dst, n) + e_self
    xlf = xl.reshape(n, HIDDEN)
    ew = jnp.repeat(e, HEAD_DIM, axis=1)
    num = _seg_sum(xlf[src] * ew, dst, n) + xlf * jnp.repeat(e_self, HEAD_DIM, axis=1)
    h = num / (jnp.repeat(denom, HEAD_DIM, axis=1) + 1e-16) + p['b_a1']
    h = _graph_norm(h, p['gn1_g'], p['gn1_b'], p['gn1_a'])
    x_emb = jax.nn.relu(h) + x_emb

    y = (x_emb @ p['W_g2']) * dinv[:, None]
    acc = _seg_sum(y[src], dst, n) + y
    h = dinv[:, None] * acc + p['b_g2']
    h = _graph_norm(h, p['gn2_g'], p['gn2_b'], p['gn2_a'])
    x_emb = jax.nn.relu(h) + x_emb

    hub_input = jnp.concatenate([x_emb, structural], axis=-1)
    h1 = jax.nn.relu(hub_input @ p['hi_W1'] + p['hi_b1'])
    h2 = jax.nn.relu(h1 @ p['hi_W2'] + p['hi_b2'])
    learned = jax.nn.sigmoid((h2 @ p['hi_W3'] + p['hi_b3']).squeeze(-1))
    combined = 0.6 * learned + 0.4 * hub_scores
    hub_feat = jnp.concatenate([x_emb, combined[:, None], hub_scores[:, None]], axis=-1)
    s1 = jax.nn.relu(hub_feat @ p['sel_W1'] + p['sel_b1'])
    hub_logits = (s1 @ p['sel_W2'] + p['sel_b2']).squeeze(-1)
    noise = EXPL * 0.1 * jax.random.normal(jax.random.key(1234), hub_logits.shape,
                                           hub_logits.dtype)
    hub_logits = hub_logits + HUB_BIAS * hub_scores + noise + 2.0 * top_hub_mask.astype(jnp.float32)

    ms = _seg_max(hub_logits, batch, BATCH)
    eh = jnp.exp(hub_logits - ms[batch])
    dh = _seg_sum(eh, batch, BATCH)
    hub_probs = eh / (dh[batch] + 1e-16)

    p1 = jax.nn.relu(hub_input @ p['pat_W1'] + p['pat_b1'])
    pattern_logits = p1 @ p['pat_W2'] + p['pat_b2'] + hub_scores[:, None] * 0.5
    pattern_probs = jax.nn.softmax(pattern_logits, axis=-1)

    cnt = _seg_sum(jnp.ones((n,), jnp.float32), batch, BATCH)
    gmean = _seg_sum(x_emb, batch, BATCH) / cnt[:, None]
    gmax = _seg_max(x_emb, batch, BATCH)
    gfeat = jnp.concatenate([gmean, gmax], axis=-1)
    target_logits = jnp.zeros_like(hub_logits)
    t1 = jax.nn.relu(gfeat @ p['term_W1'] + p['term_b1'])
    term_logits = t1 @ p['term_W2'] + p['term_b2']
    term_probs = jax.nn.softmax(term_logits, axis=-1)
    v1 = jax.nn.relu(gfeat @ p['val_W1'] + p['val_b1'])
    value = v1 @ p['val_W2'] + p['val_b2']

    return (value, hub_logits, hub_probs, pattern_logits, pattern_probs,
            target_logits, term_logits, term_probs, combined, hub_scores,
            top_hub_mask, x_emb)


# speedup vs baseline: 10.0175x; 1.3188x over previous
import functools

import jax
import jax.numpy as jnp
from jax import lax
from jax.experimental import pallas as pl
from jax.experimental.pallas import tpu as pltpu
from jax.experimental.pallas import tpu_sc as plsc

HIDDEN = 64
HEADS = 8
HEAD_DIM = 8
BATCH = 16
NODE_DIM = 7
NUM_PATTERNS = 5
HUB_BIAS = 3.0
EXPL = 0.15 * (1.0 - 0.0)

_INTERPRET = jax.default_backend() == "cpu"


_BLK = 4096


def _colmax_body(x_ref, out_ref):
    i = pl.program_id(0)
    m = jnp.max(x_ref[...][:, 4:7], axis=0, keepdims=True)

    @pl.when(i == 0)
    def _():
        out_ref[...] = m

    @pl.when(i > 0)
    def _():
        out_ref[...] = jnp.maximum(out_ref[...], m)


def _hub_body(x_ref, cm_ref, out_ref):
    x = x_ref[...]
    cm = cm_ref[...]
    fan_in = x[:, 0:1]
    fan_out = x[:, 1:2]
    degc = x[:, 2:3]
    ratio = x[:, 3:4]
    degree_score = jax.nn.sigmoid(fan_in + fan_out - 3.0)
    out_degree_score = jax.nn.sigmoid(fan_out - 2.0)
    balance = jnp.clip(1.0 - jnp.abs(ratio - 1.0), 0.0, 1.0)
    pr_s = x[:, 4:5] / (cm[0, 0] + 1e-08)
    bt_s = x[:, 5:6] / (cm[0, 1] + 1e-08)
    cl_s = x[:, 6:7] / (cm[0, 2] + 1e-08)
    hub = (0.25 * degree_score + 0.2 * out_degree_score + 0.15 * balance
           + 0.15 * pr_s + 0.1 * bt_s + 0.1 * cl_s + 0.05 * degc)
    out_ref[...] = jnp.clip(hub, 0.0, 1.0)


def _hub_scores(x_pad):
    np_ = x_pad.shape[0]
    grid = np_ // _BLK if np_ % _BLK == 0 else pl.cdiv(np_, _BLK)
    cm = pl.pallas_call(
        _colmax_body,
        grid=(grid,),
        in_specs=[pl.BlockSpec((_BLK, NODE_DIM), lambda i: (i, 0))],
        out_specs=pl.BlockSpec((1, 3), lambda i: (0, 0)),
        out_shape=jax.ShapeDtypeStruct((1, 3), jnp.float32),
        interpret=_INTERPRET,
    )(x_pad)
    out = pl.pallas_call(
        _hub_body,
        grid=(grid,),
        in_specs=[
            pl.BlockSpec((_BLK, NODE_DIM), lambda i: (i, 0)),
            pl.BlockSpec((1, 3), lambda i: (0, 0)),
        ],
        out_specs=pl.BlockSpec((_BLK, 1), lambda i: (i, 0)),
        out_shape=jax.ShapeDtypeStruct((np_, 1), jnp.float32),
        interpret=_INTERPRET,
    )(x_pad, cm)
    return out[:, 0]


def _embed_body(x_ref, we_ref, be_ref, wg_ref, xemb_ref, xw_ref):
    xe = jnp.dot(x_ref[...], we_ref[...],
                 preferred_element_type=jnp.float32) + be_ref[...]
    xemb_ref[...] = xe
    xw_ref[...] = jnp.dot(xe, wg_ref[...], preferred_element_type=jnp.float32)


def _embed(x, W_embed, b_embed, W_g0):
    n = x.shape[0]
    blk = _BLK
    grid = pl.cdiv(n, blk)
    return pl.pallas_call(
        _embed_body,
        grid=(grid,),
        in_specs=[
            pl.BlockSpec((blk, NODE_DIM), lambda i: (i, 0)),
            pl.BlockSpec((NODE_DIM, HIDDEN), lambda i: (0, 0)),
            pl.BlockSpec((1, HIDDEN), lambda i: (0, 0)),
            pl.BlockSpec((HIDDEN, HIDDEN), lambda i: (0, 0)),
        ],
        out_specs=[
            pl.BlockSpec((blk, HIDDEN), lambda i: (i, 0)),
            pl.BlockSpec((blk, HIDDEN), lambda i: (i, 0)),
        ],
        out_shape=[
            jax.ShapeDtypeStruct((n, HIDDEN), jnp.float32),
            jax.ShapeDtypeStruct((n, HIDDEN), jnp.float32),
        ],
        interpret=_INTERPRET,
    )(x, W_embed, b_embed.reshape(1, HIDDEN), W_g0)


_NS = 16
_CH = 128
_STG = 992
_HALF = HIDDEN // 2


def _degree_sc(dst, n):
    e = dst.shape[0]
    ept = e // (2 * _NS)
    nch = ept // _CH
    tail = e - 2 * _NS * nch * _CH
    ntail = pl.cdiv(tail, _CH) if tail else 0
    rpt = n // _NS
    mesh = plsc.VectorSubcoreMesh(core_axis_name="c", subcore_axis_name="s")

    assert tail % _CH == 0

    @functools.partial(
        pl.kernel, mesh=mesh,
        out_type=jax.ShapeDtypeStruct((2 * n,), jnp.float32),
        scratch_types=[
            pltpu.VMEM((_CH,), jnp.int32),
            pltpu.VMEM((_CH,), jnp.float32),
            pltpu.VMEM((_STG,), jnp.float32),
            pltpu.VMEM_SHARED((n,), jnp.float32),
        ],
    )
    def k(dst_h, out, didx, ones_v, stage, acc):
        c = lax.axis_index("c")
        s = lax.axis_index("s")

        def zfill(i, _):
            stage[pl.ds(i * 16, 16)] = jnp.zeros((16,), jnp.float32)
            return 0
        lax.fori_loop(0, _STG // 16, zfill, 0)

        base = s * rpt
        def zrow(j, _):
            pltpu.sync_copy(stage, acc.at[pl.ds(base + j * _STG, _STG)])
            return 0
        lax.fori_loop(0, rpt // _STG, zrow, 0)
        rem = rpt % _STG
        if rem:
            pltpu.sync_copy(stage.at[pl.ds(0, rem)],
                            acc.at[pl.ds(base + (rpt // _STG) * _STG, rem)])

        def fill(i, _):
            ones_v[pl.ds(i * 16, 16)] = jnp.ones((16,), jnp.float32)
            return 0
        lax.fori_loop(0, _CH // 16, fill, 0)
        plsc.subcore_barrier()

        ebase = (c * _NS + s) * (nch * _CH)
        def body(kk, _):
            off = ebase + kk * _CH
            pltpu.sync_copy(dst_h.at[pl.ds(off, _CH)], didx)
            pltpu.sync_copy(ones_v, acc.at[didx], add=True)
            return 0
        lax.fori_loop(0, nch, body, 0)

        if tail:
            @pl.when(jnp.logical_and(c == 0, s == 0))
            def _():
                def tbody(kk, _):
                    off = 2 * _NS * nch * _CH + kk * _CH
                    pltpu.sync_copy(dst_h.at[pl.ds(off, _CH)], didx)
                    pltpu.sync_copy(ones_v, acc.at[didx], add=True)
                    return 0
                lax.fori_loop(0, ntail, tbody, 0)
        plsc.subcore_barrier()

        def wout(j, _):
            roff = base + j * _STG
            pltpu.sync_copy(acc.at[pl.ds(roff, _STG)], stage)
            pltpu.sync_copy(stage, out.at[pl.ds(pl.multiple_of(c * n + roff, 8), _STG)])
            return 0
        lax.fori_loop(0, rpt // _STG, wout, 0)
        if rem:
            roff = base + (rpt // _STG) * _STG
            pltpu.sync_copy(acc.at[pl.ds(roff, rem)], stage.at[pl.ds(0, rem)])
            pltpu.sync_copy(stage.at[pl.ds(0, rem)],
                            out.at[pl.ds(pl.multiple_of(c * n + roff, 8), rem)])

    out = k(dst)
    return out[:n], out[n:]


def _seg_rows_sc(y2, src2, dst):
    n = y2.shape[0] // 2
    e = dst.shape[0]
    epw = e // _NS
    nch = epw // _CH
    rem_e = epw - nch * _CH
    rpt = n // _NS
    mesh = plsc.VectorSubcoreMesh(core_axis_name="c", subcore_axis_name="s")

    @functools.partial(
        pl.kernel, mesh=mesh,
        out_type=jax.ShapeDtypeStruct((2 * n, _HALF), jnp.float32),
        scratch_types=[
            pltpu.VMEM((_CH,), jnp.int32),
            pltpu.VMEM((_CH,), jnp.int32),
            pltpu.VMEM((_CH, _HALF), jnp.float32),
            pltpu.VMEM_SHARED((n, _HALF), jnp.float32),
            pltpu.SemaphoreType.DMA,
        ],
        compiler_params=pltpu.CompilerParams(use_tc_tiling_on_sc=False),
    )
    def k(y_h, src_h, dst_h, out, sidx, didx, rows, acc, sem):
        c = lax.axis_index("c")
        s = lax.axis_index("s")
        dummy = n - 1

        def zr(i, _):
            def zc(j, _):
                rows[i, pl.ds(j * 16, 16)] = jnp.zeros((16,), jnp.float32)
                return 0
            return lax.fori_loop(0, _HALF // 16, zc, 0)
        lax.fori_loop(0, _CH, zr, 0)

        base = s * rpt
        def zcp(j, _):
            pltpu.sync_copy(rows, acc.at[pl.ds(base + j * _CH, _CH)])
            return 0
        lax.fori_loop(0, rpt // _CH, zcp, 0)
        rem = rpt % _CH
        if rem:
            pltpu.sync_copy(rows.at[pl.ds(0, rem)],
                            acc.at[pl.ds(base + (rpt // _CH) * _CH, rem)])
        plsc.subcore_barrier()

        ebase = s * epw

        def gather_scatter():
            pltpu.async_copy(y_h.at[sidx], rows, sem).wait()
            pltpu.sync_copy(rows, acc.at[didx], add=True)

        def body(kk, _):
            off = pl.multiple_of(c * e + ebase + kk * _CH, 8)
            doff = ebase + kk * _CH
            pltpu.sync_copy(src_h.at[pl.ds(off, _CH)], sidx)
            pltpu.sync_copy(dst_h.at[pl.ds(doff, _CH)], didx)
            gather_scatter()
            return 0
        lax.fori_loop(0, nch, body, 0)

        if rem_e:
            assert rem_e % 16 == 0
            off = pl.multiple_of(c * e + ebase + nch * _CH, 8)
            doff = ebase + nch * _CH
            pltpu.sync_copy(src_h.at[pl.ds(off, rem_e)], sidx.at[pl.ds(0, rem_e)])
            pltpu.sync_copy(dst_h.at[pl.ds(doff, rem_e)], didx.at[pl.ds(0, rem_e)])
            def dfix(i, _):
                o = rem_e + i * 16
                sidx[pl.ds(o, 16)] = jnp.zeros((16,), jnp.int32)
                didx[pl.ds(o, 16)] = jnp.full((16,), dummy, jnp.int32)
                return 0
            lax.fori_loop(0, (_CH - rem_e) // 16, dfix, 0)
            gather_scatter()
        plsc.subcore_barrier()

        def wout(j, _):
            roff = base + j * _CH
            pltpu.sync_copy(acc.at[pl.ds(roff, _CH)], rows)
            pltpu.sync_copy(rows, out.at[pl.ds(pl.multiple_of(c * n + roff, 8), _CH)])
            return 0
        lax.fori_loop(0, rpt // _CH, wout, 0)
        if rem:
            roff = base + (rpt // _CH) * _CH
            pltpu.sync_copy(acc.at[pl.ds(roff, rem)], rows.at[pl.ds(0, rem)])
            pltpu.sync_copy(rows.at[pl.ds(0, rem)],
                            out.at[pl.ds(pl.multiple_of(c * n + roff, 8), rem)])

    return k(y2, src2, dst)


def _seg_sum_rows(y, src2, dst, n):
    n_pad = 50048
    y_pad = jnp.pad(y, ((0, n_pad - n), (0, 0)))
    y2 = jnp.concatenate([y_pad[:, :_HALF], y_pad[:, _HALF:]], axis=0)
    out = _seg_rows_sc(y2, src2, dst)
    return jnp.concatenate([out[:n], out[n_pad:n_pad + n]], axis=-1)


def _seg_sum(d, i, n):
    return jax.ops.segment_sum(d, i, num_segments=n)


def _seg_max(d, i, n):
    return jax.ops.segment_max(d, i, num_segments=n)


def _graph_norm(x, gamma, beta, mean_scale):
    mean = x.mean(axis=0, keepdims=True)
    centered = x - mean_scale * mean
    var = (centered * centered).mean(axis=0, keepdims=True)
    return gamma * centered / jnp.sqrt(var + 1e-05) + beta


def _top_hubs(hub_scores, batch, top_k, batch_size):
    mask = jnp.zeros(hub_scores.shape, dtype=bool)
    for b in range(batch_size):
        masked = jnp.where(batch == b, hub_scores, -jnp.inf)
        _, idx = jax.lax.top_k(masked, top_k)
        mask = mask.at[idx].set(True)
    return mask


def kernel(x, params, edge_index, batch):
    n = x.shape[0]
    p = params
    src = edge_index[0]
    dst = edge_index[1]

    structural = x[:, :NODE_DIM]
    n_pad = ((n + _BLK - 1) // _BLK) * _BLK
    x_pad = jnp.pad(x, ((0, n_pad - n), (0, 0)))
    hub_scores = _hub_scores(x_pad)[:n]
    top_hub_mask = _top_hubs(hub_scores, batch, 3, BATCH)

    x_emb, xw0 = _embed(x_pad, p['W_embed'], p['b_embed'], p['W_g0'])
    x_emb, xw0 = x_emb[:n], xw0[:n]

    dega, degb = _degree_sc(dst, 50048)
    deg = (dega + degb)[:n] + 1.0
    dinv = jax.lax.rsqrt(deg)
    src2 = jnp.concatenate([src, src + 50048])

    y = xw0 * dinv[:, None]
    acc = _seg_sum_rows(y, src2, dst, n) + y
    h = dinv[:, None] * acc + p['b_g0']
    h = _graph_norm(h, p['gn0_g'], p['gn0_b'], p['gn0_a'])
    x_emb = jax.nn.relu(h) + x_emb

    xl = (x_emb @ p['W_a1']).reshape(n, HEADS, HEAD_DIM)
    a_src = (xl * p['att_src']).sum(-1)
    a_dst = (xl * p['att_dst']).sum(-1)
    alpha_e = jax.nn.leaky_relu(a_src[src] + a_dst[dst], 0.2)
    alpha_self = jax.nn.leaky_relu(a_src + a_dst, 0.2)
    m = jnp.maximum(_seg_max(alpha_e, dst, n), alpha_self)
    e = jnp.exp(alpha_e - m[dst])
    e_self = jnp.exp(alpha_self - m)
    denom = _seg_sum(e, dst, n) + e_self
    xlf = xl.reshape(n, HIDDEN)
    ew = jnp.repeat(e, HEAD_DIM, axis=1)
    num = _seg_sum(xlf[src] * ew, dst, n) + xlf * jnp.repeat(e_self, HEAD_DIM, axis=1)
    h = num / (jnp.repeat(denom, HEAD_DIM, axis=1) + 1e-16) + p['b_a1']
    h = _graph_norm(h, p['gn1_g'], p['gn1_b'], p['gn1_a'])
    x_emb = jax.nn.relu(h) + x_emb

    y = (x_emb @ p['W_g2']) * dinv[:, None]
    acc = _seg_sum_rows(y, src2, dst, n) + y
    h = dinv[:, None] * acc + p['b_g2']
    h = _graph_norm(h, p['gn2_g'], p['gn2_b'], p['gn2_a'])
    x_emb = jax.nn.relu(h) + x_emb

    hub_input = jnp.concatenate([x_emb, structural], axis=-1)
    h1 = jax.nn.relu(hub_input @ p['hi_W1'] + p['hi_b1'])
    h2 = jax.nn.relu(h1 @ p['hi_W2'] + p['hi_b2'])
    learned = jax.nn.sigmoid((h2 @ p['hi_W3'] + p['hi_b3']).squeeze(-1))
    combined = 0.6 * learned + 0.4 * hub_scores
    hub_feat = jnp.concatenate([x_emb, combined[:, None], hub_scores[:, None]], axis=-1)
    s1 = jax.nn.relu(hub_feat @ p['sel_W1'] + p['sel_b1'])
    hub_logits = (s1 @ p['sel_W2'] + p['sel_b2']).squeeze(-1)
    noise = EXPL * 0.1 * jax.random.normal(jax.random.key(1234), hub_logits.shape,
                                           hub_logits.dtype)
    hub_logits = hub_logits + HUB_BIAS * hub_scores + noise + 2.0 * top_hub_mask.astype(jnp.float32)

    ms = _seg_max(hub_logits, batch, BATCH)
    eh = jnp.exp(hub_logits - ms[batch])
    dh = _seg_sum(eh, batch, BATCH)
    hub_probs = eh / (dh[batch] + 1e-16)

    p1 = jax.nn.relu(hub_input @ p['pat_W1'] + p['pat_b1'])
    pattern_logits = p1 @ p['pat_W2'] + p['pat_b2'] + hub_scores[:, None] * 0.5
    pattern_probs = jax.nn.softmax(pattern_logits, axis=-1)

    cnt = _seg_sum(jnp.ones((n,), jnp.float32), batch, BATCH)
    gmean = _seg_sum(x_emb, batch, BATCH) / cnt[:, None]
    gmax = _seg_max(x_emb, batch, BATCH)
    gfeat = jnp.concatenate([gmean, gmax], axis=-1)
    target_logits = jnp.zeros_like(hub_logits)
    t1 = jax.nn.relu(gfeat @ p['term_W1'] + p['term_b1'])
    term_logits = t1 @ p['term_W2'] + p['term_b2']
    term_probs = jax.nn.softmax(term_logits, axis=-1)
    v1 = jax.nn.relu(gfeat @ p['val_W1'] + p['val_b1'])
    value = v1 @ p['val_W2'] + p['val_b2']

    return (value, hub_logits, hub_probs, pattern_logits, pattern_probs,
            target_logits, term_logits, term_probs, combined, hub_scores,
            top_hub_mask, x_emb)


# speedup vs baseline: 10.0217x; 1.0004x over previous
import functools

import jax
import jax.numpy as jnp
from jax import lax
from jax.experimental import pallas as pl
from jax.experimental.pallas import tpu as pltpu
from jax.experimental.pallas import tpu_sc as plsc

HIDDEN = 64
HEADS = 8
HEAD_DIM = 8
BATCH = 16
NODE_DIM = 7
NUM_PATTERNS = 5
HUB_BIAS = 3.0
EXPL = 0.15 * (1.0 - 0.0)

_BLK = 4096


def _colmax_body(x_ref, out_ref):
    i = pl.program_id(0)
    m = jnp.max(x_ref[...][:, 4:7], axis=0, keepdims=True)

    @pl.when(i == 0)
    def _():
        out_ref[...] = m

    @pl.when(i > 0)
    def _():
        out_ref[...] = jnp.maximum(out_ref[...], m)


def _hub_body(x_ref, cm_ref, out_ref):
    x = x_ref[...]
    cm = cm_ref[...]
    fan_in = x[:, 0:1]
    fan_out = x[:, 1:2]
    degc = x[:, 2:3]
    ratio = x[:, 3:4]
    degree_score = jax.nn.sigmoid(fan_in + fan_out - 3.0)
    out_degree_score = jax.nn.sigmoid(fan_out - 2.0)
    balance = jnp.clip(1.0 - jnp.abs(ratio - 1.0), 0.0, 1.0)
    pr_s = x[:, 4:5] / (cm[0, 0] + 1e-08)
    bt_s = x[:, 5:6] / (cm[0, 1] + 1e-08)
    cl_s = x[:, 6:7] / (cm[0, 2] + 1e-08)
    hub = (0.25 * degree_score + 0.2 * out_degree_score + 0.15 * balance
           + 0.15 * pr_s + 0.1 * bt_s + 0.1 * cl_s + 0.05 * degc)
    out_ref[...] = jnp.clip(hub, 0.0, 1.0)


def _hub_scores(x_pad):
    np_ = x_pad.shape[0]
    grid = np_ // _BLK if np_ % _BLK == 0 else pl.cdiv(np_, _BLK)
    cm = pl.pallas_call(
        _colmax_body,
        grid=(grid,),
        in_specs=[pl.BlockSpec((_BLK, NODE_DIM), lambda i: (i, 0))],
        out_specs=pl.BlockSpec((1, 3), lambda i: (0, 0)),
        out_shape=jax.ShapeDtypeStruct((1, 3), jnp.float32),
    )(x_pad)
    out = pl.pallas_call(
        _hub_body,
        grid=(grid,),
        in_specs=[
            pl.BlockSpec((_BLK, NODE_DIM), lambda i: (i, 0)),
            pl.BlockSpec((1, 3), lambda i: (0, 0)),
        ],
        out_specs=pl.BlockSpec((_BLK, 1), lambda i: (i, 0)),
        out_shape=jax.ShapeDtypeStruct((np_, 1), jnp.float32),
    )(x_pad, cm)
    return out[:, 0]


def _embed_body(x_ref, we_ref, be_ref, wg_ref, xemb_ref, xw_ref):
    xe = jnp.dot(x_ref[...], we_ref[...],
                 preferred_element_type=jnp.float32) + be_ref[...]
    xemb_ref[...] = xe
    xw_ref[...] = jnp.dot(xe, wg_ref[...], preferred_element_type=jnp.float32)


def _embed(x, W_embed, b_embed, W_g0):
    n = x.shape[0]
    blk = _BLK
    grid = pl.cdiv(n, blk)
    return pl.pallas_call(
        _embed_body,
        grid=(grid,),
        in_specs=[
            pl.BlockSpec((blk, NODE_DIM), lambda i: (i, 0)),
            pl.BlockSpec((NODE_DIM, HIDDEN), lambda i: (0, 0)),
            pl.BlockSpec((1, HIDDEN), lambda i: (0, 0)),
            pl.BlockSpec((HIDDEN, HIDDEN), lambda i: (0, 0)),
        ],
        out_specs=[
            pl.BlockSpec((blk, HIDDEN), lambda i: (i, 0)),
            pl.BlockSpec((blk, HIDDEN), lambda i: (i, 0)),
        ],
        out_shape=[
            jax.ShapeDtypeStruct((n, HIDDEN), jnp.float32),
            jax.ShapeDtypeStruct((n, HIDDEN), jnp.float32),
        ],
    )(x, W_embed, b_embed.reshape(1, HIDDEN), W_g0)


_NS = 16
_CH = 128
_STG = 992
_HALF = HIDDEN // 2


def _degree_sc(dst, n):
    e = dst.shape[0]
    ept = e // (2 * _NS)
    nch = ept // _CH
    tail = e - 2 * _NS * nch * _CH
    ntail = pl.cdiv(tail, _CH) if tail else 0
    rpt = n // _NS
    mesh = plsc.VectorSubcoreMesh(core_axis_name="c", subcore_axis_name="s")

    assert tail % _CH == 0

    @functools.partial(
        pl.kernel, mesh=mesh,
        out_type=jax.ShapeDtypeStruct((2 * n,), jnp.float32),
        scratch_types=[
            pltpu.VMEM((_CH,), jnp.int32),
            pltpu.VMEM((_CH,), jnp.float32),
            pltpu.VMEM((_STG,), jnp.float32),
            pltpu.VMEM_SHARED((n,), jnp.float32),
        ],
    )
    def k(dst_h, out, didx, ones_v, stage, acc):
        c = lax.axis_index("c")
        s = lax.axis_index("s")

        def zfill(i, _):
            stage[pl.ds(i * 16, 16)] = jnp.zeros((16,), jnp.float32)
            return 0
        lax.fori_loop(0, _STG // 16, zfill, 0)

        base = s * rpt
        def zrow(j, _):
            pltpu.sync_copy(stage, acc.at[pl.ds(base + j * _STG, _STG)])
            return 0
        lax.fori_loop(0, rpt // _STG, zrow, 0)
        rem = rpt % _STG
        if rem:
            pltpu.sync_copy(stage.at[pl.ds(0, rem)],
                            acc.at[pl.ds(base + (rpt // _STG) * _STG, rem)])

        def fill(i, _):
            ones_v[pl.ds(i * 16, 16)] = jnp.ones((16,), jnp.float32)
            return 0
        lax.fori_loop(0, _CH // 16, fill, 0)
        plsc.subcore_barrier()

        ebase = (c * _NS + s) * (nch * _CH)
        def body(kk, _):
            off = ebase + kk * _CH
            pltpu.sync_copy(dst_h.at[pl.ds(off, _CH)], didx)
            pltpu.sync_copy(ones_v, acc.at[didx], add=True)
            return 0
        lax.fori_loop(0, nch, body, 0)

        if tail:
            @pl.when(jnp.logical_and(c == 0, s == 0))
            def _():
                def tbody(kk, _):
                    off = 2 * _NS * nch * _CH + kk * _CH
                    pltpu.sync_copy(dst_h.at[pl.ds(off, _CH)], didx)
                    pltpu.sync_copy(ones_v, acc.at[didx], add=True)
                    return 0
                lax.fori_loop(0, ntail, tbody, 0)
        plsc.subcore_barrier()

        def wout(j, _):
            roff = base + j * _STG
            pltpu.sync_copy(acc.at[pl.ds(roff, _STG)], stage)
            pltpu.sync_copy(stage, out.at[pl.ds(pl.multiple_of(c * n + roff, 8), _STG)])
            return 0
        lax.fori_loop(0, rpt // _STG, wout, 0)
        if rem:
            roff = base + (rpt // _STG) * _STG
            pltpu.sync_copy(acc.at[pl.ds(roff, rem)], stage.at[pl.ds(0, rem)])
            pltpu.sync_copy(stage.at[pl.ds(0, rem)],
                            out.at[pl.ds(pl.multiple_of(c * n + roff, 8), rem)])

    out = k(dst)
    return out[:n], out[n:]


def _seg_rows_sc(y2, src2, dst):
    n = y2.shape[0] // 2
    e = dst.shape[0]
    epw = e // _NS
    nch = epw // _CH
    rem_e = epw - nch * _CH
    rpt = n // _NS
    mesh = plsc.VectorSubcoreMesh(core_axis_name="c", subcore_axis_name="s")

    @functools.partial(
        pl.kernel, mesh=mesh,
        out_type=jax.ShapeDtypeStruct((2 * n, _HALF), jnp.float32),
        scratch_types=[
            pltpu.VMEM((_CH,), jnp.int32),
            pltpu.VMEM((_CH,), jnp.int32),
            pltpu.VMEM((_CH, _HALF), jnp.float32),
            pltpu.VMEM_SHARED((n, _HALF), jnp.float32),
            pltpu.SemaphoreType.DMA,
        ],
        compiler_params=pltpu.CompilerParams(use_tc_tiling_on_sc=False),
    )
    def k(y_h, src_h, dst_h, out, sidx, didx, rows, acc, sem):
        c = lax.axis_index("c")
        s = lax.axis_index("s")
        dummy = n - 1

        def zr(i, _):
            def zc(j, _):
                rows[i, pl.ds(j * 16, 16)] = jnp.zeros((16,), jnp.float32)
                return 0
            return lax.fori_loop(0, _HALF // 16, zc, 0)
        lax.fori_loop(0, _CH, zr, 0)

        base = s * rpt
        def zcp(j, _):
            pltpu.sync_copy(rows, acc.at[pl.ds(base + j * _CH, _CH)])
            return 0
        lax.fori_loop(0, rpt // _CH, zcp, 0)
        rem = rpt % _CH
        if rem:
            pltpu.sync_copy(rows.at[pl.ds(0, rem)],
                            acc.at[pl.ds(base + (rpt // _CH) * _CH, rem)])
        plsc.subcore_barrier()

        ebase = s * epw

        def gather_scatter():
            pltpu.async_copy(y_h.at[sidx], rows, sem).wait()
            pltpu.sync_copy(rows, acc.at[didx], add=True)

        def body(kk, _):
            off = pl.multiple_of(c * e + ebase + kk * _CH, 8)
            doff = ebase + kk * _CH
            pltpu.sync_copy(src_h.at[pl.ds(off, _CH)], sidx)
            pltpu.sync_copy(dst_h.at[pl.ds(doff, _CH)], didx)
            gather_scatter()
            return 0
        lax.fori_loop(0, nch, body, 0)

        if rem_e:
            assert rem_e % 16 == 0
            off = pl.multiple_of(c * e + ebase + nch * _CH, 8)
            doff = ebase + nch * _CH
            pltpu.sync_copy(src_h.at[pl.ds(off, rem_e)], sidx.at[pl.ds(0, rem_e)])
            pltpu.sync_copy(dst_h.at[pl.ds(doff, rem_e)], didx.at[pl.ds(0, rem_e)])
            def dfix(i, _):
                o = rem_e + i * 16
                sidx[pl.ds(o, 16)] = jnp.zeros((16,), jnp.int32)
                didx[pl.ds(o, 16)] = jnp.full((16,), dummy, jnp.int32)
                return 0
            lax.fori_loop(0, (_CH - rem_e) // 16, dfix, 0)
            gather_scatter()
        plsc.subcore_barrier()

        def wout(j, _):
            roff = base + j * _CH
            pltpu.sync_copy(acc.at[pl.ds(roff, _CH)], rows)
            pltpu.sync_copy(rows, out.at[pl.ds(pl.multiple_of(c * n + roff, 8), _CH)])
            return 0
        lax.fori_loop(0, rpt // _CH, wout, 0)
        if rem:
            roff = base + (rpt // _CH) * _CH
            pltpu.sync_copy(acc.at[pl.ds(roff, rem)], rows.at[pl.ds(0, rem)])
            pltpu.sync_copy(rows.at[pl.ds(0, rem)],
                            out.at[pl.ds(pl.multiple_of(c * n + roff, 8), rem)])

    return k(y2, src2, dst)


def _seg_sum_rows(y, src2, dst, n):
    n_pad = 50048
    y_pad = jnp.pad(y, ((0, n_pad - n), (0, 0)))
    y2 = jnp.concatenate([y_pad[:, :_HALF], y_pad[:, _HALF:]], axis=0)
    out = _seg_rows_sc(y2, src2, dst)
    return jnp.concatenate([out[:n], out[n_pad:n_pad + n]], axis=-1)


def _seg_sum(d, i, n):
    return jax.ops.segment_sum(d, i, num_segments=n)


def _seg_max(d, i, n):
    return jax.ops.segment_max(d, i, num_segments=n)


def _graph_norm(x, gamma, beta, mean_scale):
    mean = x.mean(axis=0, keepdims=True)
    centered = x - mean_scale * mean
    var = (centered * centered).mean(axis=0, keepdims=True)
    return gamma * centered / jnp.sqrt(var + 1e-05) + beta


def _top_hubs(hub_scores, batch, top_k, batch_size):
    mask = jnp.zeros(hub_scores.shape, dtype=bool)
    for b in range(batch_size):
        masked = jnp.where(batch == b, hub_scores, -jnp.inf)
        _, idx = jax.lax.top_k(masked, top_k)
        mask = mask.at[idx].set(True)
    return mask


def kernel(x, params, edge_index, batch):
    n = x.shape[0]
    p = params
    src = edge_index[0]
    dst = edge_index[1]

    structural = x[:, :NODE_DIM]
    n_pad = ((n + _BLK - 1) // _BLK) * _BLK
    x_pad = jnp.pad(x, ((0, n_pad - n), (0, 0)))
    hub_scores = _hub_scores(x_pad)[:n]
    top_hub_mask = _top_hubs(hub_scores, batch, 3, BATCH)

    x_emb, xw0 = _embed(x_pad, p['W_embed'], p['b_embed'], p['W_g0'])
    x_emb, xw0 = x_emb[:n], xw0[:n]

    dega, degb = _degree_sc(dst, 50048)
    deg = (dega + degb)[:n] + 1.0
    dinv = jax.lax.rsqrt(deg)
    src2 = jnp.concatenate([src, src + 50048])

    y = xw0 * dinv[:, None]
    acc = _seg_sum_rows(y, src2, dst, n) + y
    h = dinv[:, None] * acc + p['b_g0']
    h = _graph_norm(h, p['gn0_g'], p['gn0_b'], p['gn0_a'])
    x_emb = jax.nn.relu(h) + x_emb

    xl = (x_emb @ p['W_a1']).reshape(n, HEADS, HEAD_DIM)
    a_src = (xl * p['att_src']).sum(-1)
    a_dst = (xl * p['att_dst']).sum(-1)
    alpha_e = jax.nn.leaky_relu(a_src[src] + a_dst[dst], 0.2)
    alpha_self = jax.nn.leaky_relu(a_src + a_dst, 0.2)
    m = jnp.maximum(_seg_max(alpha_e, dst, n), alpha_self)
    e = jnp.exp(alpha_e - m[dst])
    e_self = jnp.exp(alpha_self - m)
    denom = _seg_sum(e, dst, n) + e_self
    xlf = xl.reshape(n, HIDDEN)
    ew = jnp.repeat(e, HEAD_DIM, axis=1)
    num = _seg_sum(xlf[src] * ew, dst, n) + xlf * jnp.repeat(e_self, HEAD_DIM, axis=1)
    h = num / (jnp.repeat(denom, HEAD_DIM, axis=1) + 1e-16) + p['b_a1']
    h = _graph_norm(h, p['gn1_g'], p['gn1_b'], p['gn1_a'])
    x_emb = jax.nn.relu(h) + x_emb

    y = (x_emb @ p['W_g2']) * dinv[:, None]
    acc = _seg_sum_rows(y, src2, dst, n) + y
    h = dinv[:, None] * acc + p['b_g2']
    h = _graph_norm(h, p['gn2_g'], p['gn2_b'], p['gn2_a'])
    x_emb = jax.nn.relu(h) + x_emb

    hub_input = jnp.concatenate([x_emb, structural], axis=-1)
    h1 = jax.nn.relu(hub_input @ p['hi_W1'] + p['hi_b1'])
    h2 = jax.nn.relu(h1 @ p['hi_W2'] + p['hi_b2'])
    learned = jax.nn.sigmoid((h2 @ p['hi_W3'] + p['hi_b3']).squeeze(-1))
    combined = 0.6 * learned + 0.4 * hub_scores
    hub_feat = jnp.concatenate([x_emb, combined[:, None], hub_scores[:, None]], axis=-1)
    s1 = jax.nn.relu(hub_feat @ p['sel_W1'] + p['sel_b1'])
    hub_logits = (s1 @ p['sel_W2'] + p['sel_b2']).squeeze(-1)
    noise = EXPL * 0.1 * jax.random.normal(jax.random.key(1234), hub_logits.shape,
                                           hub_logits.dtype)
    hub_logits = hub_logits + HUB_BIAS * hub_scores + noise + 2.0 * top_hub_mask.astype(jnp.float32)

    ms = _seg_max(hub_logits, batch, BATCH)
    eh = jnp.exp(hub_logits - ms[batch])
    dh = _seg_sum(eh, batch, BATCH)
    hub_probs = eh / (dh[batch] + 1e-16)

    p1 = jax.nn.relu(hub_input @ p['pat_W1'] + p['pat_b1'])
    pattern_logits = p1 @ p['pat_W2'] + p['pat_b2'] + hub_scores[:, None] * 0.5
    pattern_probs = jax.nn.softmax(pattern_logits, axis=-1)

    cnt = _seg_sum(jnp.ones((n,), jnp.float32), batch, BATCH)
    gmean = _seg_sum(x_emb, batch, BATCH) / cnt[:, None]
    gmax = _seg_max(x_emb, batch, BATCH)
    gfeat = jnp.concatenate([gmean, gmax], axis=-1)
    target_logits = jnp.zeros_like(hub_logits)
    t1 = jax.nn.relu(gfeat @ p['term_W1'] + p['term_b1'])
    term_logits = t1 @ p['term_W2'] + p['term_b2']
    term_probs = jax.nn.softmax(term_logits, axis=-1)
    v1 = jax.nn.relu(gfeat @ p['val_W1'] + p['val_b1'])
    value = v1 @ p['val_W2'] + p['val_b2']

    return (value, hub_logits, hub_probs, pattern_logits, pattern_probs,
            target_logits, term_logits, term_probs, combined, hub_scores,
            top_hub_mask, x_emb)
